# baseline (speedup 1.0000x reference)
import functools

import jax
import jax.numpy as jnp
from jax.experimental import pallas as pl
from jax.experimental.pallas import tpu as pltpu


def _body(x_ref, mt_ref, w_ref, bv_ref, fc_ref, out_ref, pt_ref, t_ref,
          g_ref, *, kk, filt, nfp, nfr, tb):
    @pl.when(pl.program_id(0) == 0)
    def _build_pt():
        mv = mt_ref[...]
        r = jax.lax.broadcasted_iota(jnp.int32, (nfp, nfp), 0)
        c = jax.lax.broadcasted_iota(jnp.int32, (nfp, nfp), 1)
        t0 = (r == c).astype(jnp.float32)
        pt_ref[0, :, :] = t0.astype(jnp.bfloat16)
        pt_ref[1, :, :] = mv.astype(jnp.bfloat16)
        t1 = mv
        for k in range(2, kk):
            t2 = 2.0 * jax.lax.dot(mv, t1, precision=jax.lax.Precision.HIGHEST,
                                   preferred_element_type=jnp.float32) - t0
            pt_ref[k, :, :] = t2.astype(jnp.bfloat16)
            t0, t1 = t1, t2

    xb = x_ref[...].astype(jnp.bfloat16)
    t_ref[0, :, :] = x_ref[...]
    for k in range(1, kk):
        t_ref[k, :, :] = jax.lax.dot(pt_ref[k, :, :], xb,
                                     preferred_element_type=jnp.float32)

    def fbody(f, carry):
        acc = t_ref[0, :nfr, :] * w_ref[0, f]
        for k in range(1, kk):
            acc = acc + t_ref[k, :nfr, :] * w_ref[k, f]
        gv = jnp.maximum(acc + bv_ref[f], 0.0)
        g_ref[pl.ds(f * nfr, nfr), :] = gv.astype(jnp.bfloat16)
        return carry

    jax.lax.fori_loop(0, filt, fbody, 0)
    out_ref[...] = jax.lax.dot(fc_ref[...], g_ref[...],
                               preferred_element_type=jnp.float32)


def kernel(x, L, W, b, fc_w, fc_b, y):
    B, N, F = x.shape
    K, FILT = W.shape
    C = fc_w.shape[1]
    NF = N * F
    NFP = 384
    NFR = 320
    TB = 512
    GROWS = FILT * NFR

    xT = jnp.pad(x.reshape(B, NF).T, ((0, NFP - NF), (0, 0)))
    Mt = jnp.kron(L, jnp.eye(F, dtype=L.dtype))
    Mtp = jnp.pad(Mt, ((0, NFP - NF), (0, NFP - NF)))
    bvec = b.reshape(FILT)
    fc3 = jnp.pad(fc_w.reshape(FILT, NF, C), ((0, 0), (0, NFR - NF), (0, 0)))
    fcT = jnp.pad(fc3.transpose(2, 0, 1).reshape(C, GROWS),
                  ((0, 8 - C), (0, 0))).astype(jnp.bfloat16)

    body = functools.partial(_body, kk=K, filt=FILT, nfp=NFP, nfr=NFR, tb=TB)
    out = pl.pallas_call(
        body,
        grid=(B // TB,),
        in_specs=[
            pl.BlockSpec((NFP, TB), lambda i: (0, i)),
            pl.BlockSpec((NFP, NFP), lambda i: (0, 0)),
            pl.BlockSpec(memory_space=pltpu.SMEM),
            pl.BlockSpec(memory_space=pltpu.SMEM),
            pl.BlockSpec((8, GROWS), lambda i: (0, 0)),
        ],
        out_specs=pl.BlockSpec((8, TB), lambda i: (0, i)),
        out_shape=jax.ShapeDtypeStruct((8, B), jnp.float32),
        scratch_shapes=[
            pltpu.VMEM((K, NFP, NFP), jnp.bfloat16),
            pltpu.VMEM((K, NFP, TB), jnp.float32),
            pltpu.VMEM((GROWS, TB), jnp.bfloat16),
        ],
        compiler_params=pltpu.CompilerParams(
            dimension_semantics=("arbitrary",)),
    )(xT, Mtp, W, bvec, fcT)
    return out[:C, :].T + fc_b[None, :]
